# Initial kernel scaffold; baseline (speedup 1.0000x reference)
import jax
import jax.numpy as jnp
from jax.experimental import pallas as pl


def kernel(sample, et, entity_embedding, relation_embedding):
    raise NotImplementedError("write your pallas kernel here")


# speedup vs baseline: 1.2488x; 1.2488x over previous
import functools

import jax
import jax.numpy as jnp
from jax import lax
from jax.experimental import pallas as pl
from jax.experimental.pallas import tpu as pltpu
from jax.experimental.pallas import tpu_sc as plsc

GAMMA = 12.0
HIDDEN = 128
ENT_DIM = 256
BATCH = 16384
EMB_RANGE = (12.0 + 2.0) / HIDDEN
PHASE_K = 3.141592653589793 / EMB_RANGE

NUM_WORKERS = 32
BPW = BATCH // NUM_WORKERS
CHUNK = 128
NCHUNK = BPW // CHUNK
NGRP = CHUNK // 16
NDC = HIDDEN // 16

SIN_C = (0.9999997061901769, -0.16666577004061023, 0.008332556662499432,
         -0.00019812535509900264, 2.704003925453127e-06,
         -2.053224125910167e-08)
COS_C = (0.999999442124411, -0.4999955718715294, 0.041661022968180556,
         -0.0013862713619059118, 2.4252730241512727e-05,
         -2.2191767715371245e-07)
RSQRT_MAGIC = 0x5F3759DF


def _poly_even(x2, coeffs):
    acc = jnp.full((16,), coeffs[-1], jnp.float32)
    for c in coeffs[-2::-1]:
        acc = acc * x2 + c
    return acc


def _sqrt16(v):
    bits = lax.bitcast_convert_type(v, jnp.int32)
    seed = RSQRT_MAGIC - lax.shift_right_arithmetic(bits, 1)
    y = lax.bitcast_convert_type(seed, jnp.float32)
    half_v = 0.5 * v
    for _ in range(3):
        y = y * (1.5 - half_v * y * y)
    return v * y


def _make_sc_kernel():
    mesh = plsc.VectorSubcoreMesh(core_axis_name="c", subcore_axis_name="s")

    @functools.partial(
        pl.kernel,
        mesh=mesh,
        compiler_params=pltpu.CompilerParams(needs_layout_passes=False),
        out_type=jax.ShapeDtypeStruct((BATCH,), jnp.float32),
        scratch_types=[
            pltpu.VMEM((CHUNK,), jnp.int32),
            pltpu.VMEM((CHUNK,), jnp.int32),
            pltpu.VMEM((CHUNK,), jnp.int32),
            pltpu.VMEM((CHUNK, ENT_DIM), jnp.float32),
            pltpu.VMEM((CHUNK, ENT_DIM), jnp.float32),
            pltpu.VMEM((CHUNK, HIDDEN), jnp.float32),
            pltpu.VMEM((CHUNK,), jnp.float32),
            pltpu.VMEM((16,), jnp.float32),
            pltpu.SemaphoreType.DMA,
            pltpu.SemaphoreType.DMA,
            pltpu.SemaphoreType.DMA,
        ],
    )
    def rotate_score(hidx_hbm, tidx_hbm, et_hbm, ent_hbm, rel_hbm, out_hbm,
                     hidx_v, tidx_v, et_v, head_v, tail_v, rel_v,
                     out_v, red_v, sem_h, sem_t, sem_r):
        wid = lax.axis_index("s") * 2 + lax.axis_index("c")
        base = wid * BPW
        lane_iota = lax.broadcasted_iota(jnp.int32, (16,), 0)

        for c in range(NCHUNK):
            cb = base + c * CHUNK
            pltpu.sync_copy(hidx_hbm.at[pl.ds(cb, CHUNK)], hidx_v)
            pltpu.sync_copy(tidx_hbm.at[pl.ds(cb, CHUNK)], tidx_v)
            pltpu.sync_copy(et_hbm.at[pl.ds(cb, CHUNK)], et_v)
            cp_h = pltpu.async_copy(ent_hbm.at[hidx_v], head_v, sem_h)
            cp_t = pltpu.async_copy(ent_hbm.at[tidx_v], tail_v, sem_t)
            cp_r = pltpu.async_copy(rel_hbm.at[et_v], rel_v, sem_r)
            cp_h.wait()
            cp_t.wait()
            cp_r.wait()

            for g in range(NGRP):
                def elem_body(e, gacc, g=g):
                    i = g * 16 + e
                    acc = jnp.zeros((16,), jnp.float32)
                    for dc in range(NDC):
                        sl = pl.ds(dc * 16, 16)
                        sl_im = pl.ds(HIDDEN + dc * 16, 16)
                        ph = rel_v[i, sl] * PHASE_K
                        x2 = ph * ph
                        sin_r = _poly_even(x2, SIN_C) * ph
                        cos_r = _poly_even(x2, COS_C)
                        re_h = head_v[i, sl]
                        im_h = head_v[i, sl_im]
                        re_s = re_h * cos_r - im_h * sin_r - tail_v[i, sl]
                        im_s = re_h * sin_r + im_h * cos_r - tail_v[i, sl_im]
                        acc = acc + _sqrt16(re_s * re_s + im_s * im_s)
                    for sh in (8, 4, 2, 1):
                        red_v[...] = acc
                        idx = jnp.bitwise_xor(lane_iota, sh)
                        acc = acc + plsc.load_gather(red_v, [idx])
                    return jnp.where(lane_iota == e, acc, gacc)

                tot = lax.fori_loop(0, 16, elem_body,
                                    jnp.zeros((16,), jnp.float32))
                sc = GAMMA - tot
                u = jnp.exp(-jnp.abs(sc))
                t = u / (u + 2.0)
                t2 = t * t
                log1p = 2.0 * t * (1.0 + t2 * (1.0 / 3.0 + t2 * (
                    1.0 / 5.0 + t2 * (1.0 / 7.0 + t2 * (1.0 / 9.0)))))
                out_v[pl.ds(g * 16, 16)] = jnp.minimum(sc, 0.0) - log1p
            pltpu.sync_copy(out_v, out_hbm.at[pl.ds(cb, CHUNK)])

    return rotate_score


_SC_KERNEL = _make_sc_kernel()


@jax.jit
def kernel(sample, et, entity_embedding, relation_embedding):
    head_idx = sample[0]
    tail_idx = sample[1]
    return _SC_KERNEL(head_idx, tail_idx, et,
                      entity_embedding, relation_embedding)


# speedup vs baseline: 1.4789x; 1.1843x over previous
import functools

import jax
import jax.numpy as jnp
from jax import lax
from jax.experimental import pallas as pl
from jax.experimental.pallas import tpu as pltpu
from jax.experimental.pallas import tpu_sc as plsc

GAMMA = 12.0
HIDDEN = 128
ENT_DIM = 256
BATCH = 16384
EMB_RANGE = (12.0 + 2.0) / HIDDEN
PHASE_K = 3.141592653589793 / EMB_RANGE

NUM_WORKERS = 32
BPW = BATCH // NUM_WORKERS
CHUNK = 128
NCHUNK = BPW // CHUNK
NGRP = CHUNK // 16
NDC = HIDDEN // 16
RED_STRIDE = 17

SIN_C = (0.9999845572044254, -0.16663253012127177, 0.008312359265058881,
         -0.00019315794001690636, 2.17300252207157e-06)
COS_C = (0.9999710254016718, -0.49983729380023234, 0.04152210550229491,
         -0.0013440665390150703, 1.9062735626078882e-05)
RSQRT_MAGIC = 0x5F3759DF


def _poly_even(x2, coeffs):
    acc = jnp.full((16,), coeffs[-1], jnp.float32)
    for c in coeffs[-2::-1]:
        acc = acc * x2 + c
    return acc


def _sqrt16(v):
    bits = lax.bitcast_convert_type(v, jnp.int32)
    seed = RSQRT_MAGIC - lax.shift_right_arithmetic(bits, 1)
    y = lax.bitcast_convert_type(seed, jnp.float32)
    half_v = 0.5 * v
    for _ in range(2):
        y = y * (1.5 - half_v * y * y)
    return v * y


def _make_sc_kernel():
    mesh = plsc.VectorSubcoreMesh(core_axis_name="c", subcore_axis_name="s")

    @functools.partial(
        pl.kernel,
        mesh=mesh,
        compiler_params=pltpu.CompilerParams(needs_layout_passes=False),
        out_type=jax.ShapeDtypeStruct((BATCH,), jnp.float32),
        scratch_types=[
            pltpu.VMEM((CHUNK,), jnp.int32),
            pltpu.VMEM((CHUNK,), jnp.int32),
            pltpu.VMEM((CHUNK,), jnp.int32),
            pltpu.VMEM((CHUNK, ENT_DIM), jnp.float32),
            pltpu.VMEM((CHUNK, ENT_DIM), jnp.float32),
            pltpu.VMEM((CHUNK, HIDDEN), jnp.float32),
            pltpu.VMEM((CHUNK,), jnp.float32),
            pltpu.VMEM((16 * RED_STRIDE,), jnp.float32),
            pltpu.SemaphoreType.DMA,
            pltpu.SemaphoreType.DMA,
            pltpu.SemaphoreType.DMA,
        ],
    )
    def rotate_score(sample_hbm, et_hbm, ent_hbm, rel_hbm, out_hbm,
                     hidx_v, tidx_v, et_v, head_v, tail_v, rel_v,
                     out_v, red_v, sem_h, sem_t, sem_r):
        wid = lax.axis_index("s") * 2 + lax.axis_index("c")
        base = wid * BPW
        lane_iota = lax.broadcasted_iota(jnp.int32, (16,), 0)
        col_base = lane_iota * RED_STRIDE

        for c in range(NCHUNK):
            cb = base + c * CHUNK
            pltpu.sync_copy(sample_hbm.at[0, pl.ds(cb, CHUNK)], hidx_v)
            pltpu.sync_copy(sample_hbm.at[1, pl.ds(cb, CHUNK)], tidx_v)
            pltpu.sync_copy(et_hbm.at[pl.ds(cb, CHUNK)], et_v)
            cp_h = pltpu.async_copy(ent_hbm.at[hidx_v], head_v, sem_h)
            cp_t = pltpu.async_copy(ent_hbm.at[tidx_v], tail_v, sem_t)
            cp_r = pltpu.async_copy(rel_hbm.at[et_v], rel_v, sem_r)
            cp_h.wait()
            cp_t.wait()
            cp_r.wait()

            for g in range(NGRP):
                def elem_body(e, carry, g=g):
                    i = g * 16 + e
                    acc = jnp.zeros((16,), jnp.float32)
                    for dc in range(NDC):
                        sl = pl.ds(dc * 16, 16)
                        sl_im = pl.ds(HIDDEN + dc * 16, 16)
                        ph = rel_v[i, sl] * PHASE_K
                        x2 = ph * ph
                        sin_r = _poly_even(x2, SIN_C) * ph
                        cos_r = _poly_even(x2, COS_C)
                        re_h = head_v[i, sl]
                        im_h = head_v[i, sl_im]
                        re_s = re_h * cos_r - im_h * sin_r - tail_v[i, sl]
                        im_s = re_h * sin_r + im_h * cos_r - tail_v[i, sl_im]
                        acc = acc + _sqrt16(re_s * re_s + im_s * im_s)
                    plsc.store_scatter(red_v, [lane_iota + e * RED_STRIDE],
                                       acc)
                    return carry

                lax.fori_loop(0, 16, elem_body, 0)
                tot = jnp.zeros((16,), jnp.float32)
                for cc in range(16):
                    tot = tot + plsc.load_gather(red_v, [col_base + cc])
                sc = GAMMA - tot
                u = jnp.exp(-jnp.abs(sc))
                t = u / (u + 2.0)
                t2 = t * t
                log1p = 2.0 * t * (1.0 + t2 * (1.0 / 3.0 + t2 * (
                    1.0 / 5.0 + t2 * (1.0 / 7.0))))
                out_v[pl.ds(g * 16, 16)] = jnp.minimum(sc, 0.0) - log1p
            pltpu.sync_copy(out_v, out_hbm.at[pl.ds(cb, CHUNK)])

    return rotate_score


_SC_KERNEL = _make_sc_kernel()


@jax.jit
def kernel(sample, et, entity_embedding, relation_embedding):
    return _SC_KERNEL(sample, et, entity_embedding, relation_embedding)


# speedup vs baseline: 1.8876x; 1.2764x over previous
import functools

import jax
import jax.numpy as jnp
from jax import lax
from jax.experimental import pallas as pl
from jax.experimental.pallas import tpu as pltpu
from jax.experimental.pallas import tpu_sc as plsc

GAMMA = 12.0
HIDDEN = 128
ENT_DIM = 256
BATCH = 16384
EMB_RANGE = (12.0 + 2.0) / HIDDEN
PHASE_K = 3.141592653589793 / EMB_RANGE

NUM_WORKERS = 32
BPW = BATCH // NUM_WORKERS
CHUNK = 128
NCHUNK = BPW // CHUNK
NGRP = CHUNK // 16
NDC = HIDDEN // 16
RED_STRIDE = 17

NREL = 1000
RSQRT_MAGIC = 0x5F3759DF


def _sincos_tc_kernel(rel_ref, out_ref):
    ph = rel_ref[...] * PHASE_K
    out_ref[:, :HIDDEN] = jnp.cos(ph)
    out_ref[:, HIDDEN:] = jnp.sin(ph)


_sincos_table = pl.pallas_call(
    _sincos_tc_kernel,
    out_shape=jax.ShapeDtypeStruct((NREL, 2 * HIDDEN), jnp.float32),
)


def _sqrt16(v):
    bits = lax.bitcast_convert_type(v, jnp.int32)
    seed = RSQRT_MAGIC - lax.shift_right_arithmetic(bits, 1)
    y = lax.bitcast_convert_type(seed, jnp.float32)
    half_v = 0.5 * v
    for _ in range(2):
        y = y * (1.5 - half_v * y * y)
    return v * y


def _make_sc_kernel():
    mesh = plsc.VectorSubcoreMesh(core_axis_name="c", subcore_axis_name="s")

    @functools.partial(
        pl.kernel,
        mesh=mesh,
        compiler_params=pltpu.CompilerParams(needs_layout_passes=False),
        out_type=jax.ShapeDtypeStruct((BATCH,), jnp.float32),
        scratch_types=[
            pltpu.VMEM((CHUNK,), jnp.int32),
            pltpu.VMEM((CHUNK,), jnp.int32),
            pltpu.VMEM((CHUNK,), jnp.int32),
            pltpu.VMEM((CHUNK, ENT_DIM), jnp.float32),
            pltpu.VMEM((CHUNK, ENT_DIM), jnp.float32),
            pltpu.VMEM((CHUNK, ENT_DIM), jnp.float32),
            pltpu.VMEM((CHUNK,), jnp.float32),
            pltpu.VMEM((16 * RED_STRIDE,), jnp.float32),
            pltpu.SemaphoreType.DMA,
            pltpu.SemaphoreType.DMA,
            pltpu.SemaphoreType.DMA,
        ],
    )
    def rotate_score(sample_hbm, et_hbm, ent_hbm, rel_hbm, out_hbm,
                     hidx_v, tidx_v, et_v, head_v, tail_v, rel_v,
                     out_v, red_v, sem_h, sem_t, sem_r):
        wid = lax.axis_index("s") * 2 + lax.axis_index("c")
        base = wid * BPW
        lane_iota = lax.broadcasted_iota(jnp.int32, (16,), 0)
        col_base = lane_iota * RED_STRIDE

        def chunk_body(c, carry0):
            cb = base + c * CHUNK
            pltpu.sync_copy(sample_hbm.at[0, pl.ds(cb, CHUNK)], hidx_v)
            pltpu.sync_copy(sample_hbm.at[1, pl.ds(cb, CHUNK)], tidx_v)
            pltpu.sync_copy(et_hbm.at[pl.ds(cb, CHUNK)], et_v)
            cp_h = pltpu.async_copy(ent_hbm.at[hidx_v], head_v, sem_h)
            cp_t = pltpu.async_copy(ent_hbm.at[tidx_v], tail_v, sem_t)
            cp_r = pltpu.async_copy(rel_hbm.at[et_v], rel_v, sem_r)
            cp_h.wait()
            cp_t.wait()
            cp_r.wait()

            def group_body(g, carry1):
                def elem_body(e, carry2):
                    i = g * 16 + e
                    acc = jnp.zeros((16,), jnp.float32)
                    for dc in range(NDC):
                        sl = pl.ds(dc * 16, 16)
                        sl_im = pl.ds(HIDDEN + dc * 16, 16)
                        cos_r = rel_v[i, sl]
                        sin_r = rel_v[i, sl_im]
                        re_h = head_v[i, sl]
                        im_h = head_v[i, sl_im]
                        re_s = re_h * cos_r - im_h * sin_r - tail_v[i, sl]
                        im_s = re_h * sin_r + im_h * cos_r - tail_v[i, sl_im]
                        acc = acc + _sqrt16(re_s * re_s + im_s * im_s)
                    plsc.store_scatter(red_v, [lane_iota + e * RED_STRIDE],
                                       acc)
                    return carry2

                lax.fori_loop(0, 16, elem_body, 0)
                tot = jnp.zeros((16,), jnp.float32)
                for cc in range(16):
                    tot = tot + plsc.load_gather(red_v, [col_base + cc])
                sc = GAMMA - tot
                u = jnp.exp(-jnp.abs(sc))
                t = u / (u + 2.0)
                t2 = t * t
                log1p = 2.0 * t * (1.0 + t2 * (1.0 / 3.0 + t2 * (
                    1.0 / 5.0 + t2 * (1.0 / 7.0))))
                out_v[pl.ds(g * 16, 16)] = jnp.minimum(sc, 0.0) - log1p
                return carry1

            lax.fori_loop(0, NGRP, group_body, 0)
            pltpu.sync_copy(out_v, out_hbm.at[pl.ds(cb, CHUNK)])
            return carry0

        lax.fori_loop(0, NCHUNK, chunk_body, 0)

    return rotate_score


_SC_KERNEL = _make_sc_kernel()


@jax.jit
def kernel(sample, et, entity_embedding, relation_embedding):
    sincos = _sincos_table(relation_embedding)
    return _SC_KERNEL(sample, et, entity_embedding, sincos)


# speedup vs baseline: 2.4658x; 1.3063x over previous
import functools

import jax
import jax.numpy as jnp
from jax import lax
from jax.experimental import pallas as pl
from jax.experimental.pallas import tpu as pltpu
from jax.experimental.pallas import tpu_sc as plsc

GAMMA = 12.0
HIDDEN = 128
ENT_DIM = 256
BATCH = 16384
EMB_RANGE = (12.0 + 2.0) / HIDDEN
PHASE_K = 3.141592653589793 / EMB_RANGE

NUM_WORKERS = 32
BPW = BATCH // NUM_WORKERS
CHUNK = 64
NCHUNK = BPW // CHUNK
NPAIR = NCHUNK // 2
NGRP = CHUNK // 16
NDC = HIDDEN // 16
RED_STRIDE = 17

NREL = 1000
RSQRT_MAGIC = 0x5F3759DF


def _sincos_tc_kernel(rel_ref, out_ref):
    ph = rel_ref[...] * PHASE_K
    out_ref[:, :HIDDEN] = jnp.cos(ph)
    out_ref[:, HIDDEN:] = jnp.sin(ph)


_sincos_table = pl.pallas_call(
    _sincos_tc_kernel,
    out_shape=jax.ShapeDtypeStruct((NREL, 2 * HIDDEN), jnp.float32),
)


def _sqrt16(v):
    bits = lax.bitcast_convert_type(v, jnp.int32)
    seed = RSQRT_MAGIC - lax.shift_right_arithmetic(bits, 1)
    y = lax.bitcast_convert_type(seed, jnp.float32)
    half_v = 0.5 * v
    for _ in range(2):
        y = y * (1.5 - half_v * y * y)
    return v * y


def _make_sc_kernel():
    mesh = plsc.VectorSubcoreMesh(core_axis_name="c", subcore_axis_name="s")

    @functools.partial(
        pl.kernel,
        mesh=mesh,
        compiler_params=pltpu.CompilerParams(needs_layout_passes=False),
        out_type=jax.ShapeDtypeStruct((BATCH,), jnp.float32),
        scratch_types=[
            pltpu.VMEM((CHUNK,), jnp.int32),
            pltpu.VMEM((CHUNK,), jnp.int32),
            pltpu.VMEM((CHUNK,), jnp.int32),
            pltpu.VMEM((CHUNK,), jnp.int32),
            pltpu.VMEM((CHUNK,), jnp.int32),
            pltpu.VMEM((CHUNK,), jnp.int32),
            pltpu.VMEM((CHUNK, ENT_DIM), jnp.float32),
            pltpu.VMEM((CHUNK, ENT_DIM), jnp.float32),
            pltpu.VMEM((CHUNK, ENT_DIM), jnp.float32),
            pltpu.VMEM((CHUNK, ENT_DIM), jnp.float32),
            pltpu.VMEM((CHUNK, ENT_DIM), jnp.float32),
            pltpu.VMEM((CHUNK, ENT_DIM), jnp.float32),
            pltpu.VMEM((CHUNK,), jnp.float32),
            pltpu.VMEM((16 * RED_STRIDE,), jnp.float32),
            pltpu.SemaphoreType.DMA,
            pltpu.SemaphoreType.DMA,
            pltpu.SemaphoreType.DMA,
            pltpu.SemaphoreType.DMA,
            pltpu.SemaphoreType.DMA,
            pltpu.SemaphoreType.DMA,
            pltpu.SemaphoreType.DMA,
            pltpu.SemaphoreType.DMA,
            pltpu.SemaphoreType.DMA,
        ],
    )
    def rotate_score(sample_hbm, et_hbm, ent_hbm, rel_hbm, out_hbm,
                     hidx0, tidx0, et0, hidx1, tidx1, et1,
                     head0, tail0, rel0, head1, tail1, rel1,
                     out_v, red_v, sem_ih, sem_it, sem_ir,
                     sem_h0, sem_t0, sem_r0, sem_h1, sem_t1, sem_r1):
        wid = lax.axis_index("s") * 2 + lax.axis_index("c")
        base = wid * BPW
        lane_iota = lax.broadcasted_iota(jnp.int32, (16,), 0)
        col_base = lane_iota * RED_STRIDE

        bufs = (
            (hidx0, tidx0, et0, head0, tail0, rel0, sem_h0, sem_t0, sem_r0),
            (hidx1, tidx1, et1, head1, tail1, rel1, sem_h1, sem_t1, sem_r1),
        )

        def issue(c, b):
            hidx_v, tidx_v, et_v, head_v, tail_v, rel_v, sh, st, sr = bufs[b]
            cb = base + c * CHUNK
            ci = pltpu.async_copy(
                sample_hbm.at[0, pl.ds(cb, CHUNK)], hidx_v, sem_ih)
            ct = pltpu.async_copy(
                sample_hbm.at[1, pl.ds(cb, CHUNK)], tidx_v, sem_it)
            ce = pltpu.async_copy(et_hbm.at[pl.ds(cb, CHUNK)], et_v, sem_ir)
            ci.wait()
            ct.wait()
            ce.wait()
            pltpu.async_copy(ent_hbm.at[hidx_v], head_v, sh)
            pltpu.async_copy(ent_hbm.at[tidx_v], tail_v, st)
            pltpu.async_copy(rel_hbm.at[et_v], rel_v, sr)

        def drain(b):
            _, _, _, head_v, tail_v, rel_v, sh, st, sr = bufs[b]
            pltpu.make_async_copy(
                ent_hbm.at[pl.ds(0, CHUNK)], head_v, sh).wait()
            pltpu.make_async_copy(
                ent_hbm.at[pl.ds(0, CHUNK)], tail_v, st).wait()
            pltpu.make_async_copy(
                rel_hbm.at[pl.ds(0, CHUNK)], rel_v, sr).wait()

        def compute(c, b):
            _, _, _, head_v, tail_v, rel_v, _, _, _ = bufs[b]
            cb = base + c * CHUNK

            def group_body(g, carry1):
                def elem_body(e, carry2):
                    i = g * 16 + e
                    acc = jnp.zeros((16,), jnp.float32)
                    for dc in range(NDC):
                        sl = pl.ds(dc * 16, 16)
                        sl_im = pl.ds(HIDDEN + dc * 16, 16)
                        cos_r = rel_v[i, sl]
                        sin_r = rel_v[i, sl_im]
                        re_h = head_v[i, sl]
                        im_h = head_v[i, sl_im]
                        re_s = re_h * cos_r - im_h * sin_r - tail_v[i, sl]
                        im_s = re_h * sin_r + im_h * cos_r - tail_v[i, sl_im]
                        acc = acc + _sqrt16(re_s * re_s + im_s * im_s)
                    plsc.store_scatter(red_v, [lane_iota + e * RED_STRIDE],
                                       acc)
                    return carry2

                lax.fori_loop(0, 16, elem_body, 0)
                tot = jnp.zeros((16,), jnp.float32)
                for cc in range(16):
                    tot = tot + plsc.load_gather(red_v, [col_base + cc])
                sc = GAMMA - tot
                u = jnp.exp(-jnp.abs(sc))
                t = u / (u + 2.0)
                t2 = t * t
                log1p = 2.0 * t * (1.0 + t2 * (1.0 / 3.0 + t2 * (
                    1.0 / 5.0 + t2 * (1.0 / 7.0))))
                out_v[pl.ds(g * 16, 16)] = jnp.minimum(sc, 0.0) - log1p
                return carry1

            lax.fori_loop(0, NGRP, group_body, 0)
            pltpu.sync_copy(out_v, out_hbm.at[pl.ds(cb, CHUNK)])

        issue(0, 0)

        def pair_body(k, carry):
            c0 = 2 * k
            issue(c0 + 1, 1)
            drain(0)
            compute(c0, 0)
            issue(c0 + 2, 0)
            drain(1)
            compute(c0 + 1, 1)
            return carry

        lax.fori_loop(0, NPAIR - 1, pair_body, 0)
        issue(NCHUNK - 1, 1)
        drain(0)
        compute(NCHUNK - 2, 0)
        drain(1)
        compute(NCHUNK - 1, 1)

    return rotate_score


_SC_KERNEL = _make_sc_kernel()


@jax.jit
def kernel(sample, et, entity_embedding, relation_embedding):
    sincos = _sincos_table(relation_embedding)
    return _SC_KERNEL(sample, et, entity_embedding, sincos)


# speedup vs baseline: 2.5814x; 1.0469x over previous
import functools

import jax
import jax.numpy as jnp
from jax import lax
from jax.experimental import pallas as pl
from jax.experimental.pallas import tpu as pltpu
from jax.experimental.pallas import tpu_sc as plsc

GAMMA = 12.0
HIDDEN = 128
ENT_DIM = 256
BATCH = 16384
EMB_RANGE = (12.0 + 2.0) / HIDDEN
PHASE_K = 3.141592653589793 / EMB_RANGE

NUM_WORKERS = 32
BPW = BATCH // NUM_WORKERS
CHUNK = 64
NCHUNK = BPW // CHUNK
NPAIR = NCHUNK // 2
NGRP = CHUNK // 16
NDC = HIDDEN // 16
RED_STRIDE = 17

NREL = 1000
RSQRT_MAGIC = 0x5F3759DF


def _sincos_tc_kernel(rel_ref, out_ref):
    ph = rel_ref[...] * PHASE_K
    out_ref[:, :HIDDEN] = jnp.cos(ph)
    out_ref[:, HIDDEN:] = jnp.sin(ph)


_sincos_table = pl.pallas_call(
    _sincos_tc_kernel,
    out_shape=jax.ShapeDtypeStruct((NREL, 2 * HIDDEN), jnp.float32),
)


def _sqrt16(v):
    bits = lax.bitcast_convert_type(v, jnp.int32)
    seed = RSQRT_MAGIC - lax.shift_right_arithmetic(bits, 1)
    y = lax.bitcast_convert_type(seed, jnp.float32)
    half_v = 0.5 * v
    for _ in range(2):
        y = y * (1.5 - half_v * y * y)
    return v * y


def _make_sc_kernel():
    mesh = plsc.VectorSubcoreMesh(core_axis_name="c", subcore_axis_name="s")

    @functools.partial(
        pl.kernel,
        mesh=mesh,
        compiler_params=pltpu.CompilerParams(needs_layout_passes=False),
        out_type=jax.ShapeDtypeStruct((BATCH,), jnp.float32),
        scratch_types=[
            pltpu.VMEM((BPW,), jnp.int32),
            pltpu.VMEM((BPW,), jnp.int32),
            pltpu.VMEM((BPW,), jnp.int32),
            pltpu.VMEM((CHUNK, ENT_DIM), jnp.float32),
            pltpu.VMEM((CHUNK, ENT_DIM), jnp.float32),
            pltpu.VMEM((CHUNK, ENT_DIM), jnp.float32),
            pltpu.VMEM((CHUNK, ENT_DIM), jnp.float32),
            pltpu.VMEM((CHUNK, ENT_DIM), jnp.float32),
            pltpu.VMEM((CHUNK, ENT_DIM), jnp.float32),
            pltpu.VMEM((CHUNK,), jnp.float32),
            pltpu.VMEM((16 * RED_STRIDE,), jnp.float32),
            pltpu.SemaphoreType.DMA,
            pltpu.SemaphoreType.DMA,
            pltpu.SemaphoreType.DMA,
            pltpu.SemaphoreType.DMA,
            pltpu.SemaphoreType.DMA,
            pltpu.SemaphoreType.DMA,
            pltpu.SemaphoreType.DMA,
            pltpu.SemaphoreType.DMA,
            pltpu.SemaphoreType.DMA,
        ],
    )
    def rotate_score(sample_hbm, et_hbm, ent_hbm, rel_hbm, out_hbm,
                     hidx_all, tidx_all, eidx_all,
                     head0, tail0, rel0, head1, tail1, rel1,
                     out_v, red_v, sem_ih, sem_it, sem_ir,
                     sem_h0, sem_t0, sem_r0, sem_h1, sem_t1, sem_r1):
        wid = lax.axis_index("s") * 2 + lax.axis_index("c")
        base = wid * BPW
        lane_iota = lax.broadcasted_iota(jnp.int32, (16,), 0)
        col_base = lane_iota * RED_STRIDE

        ci = pltpu.async_copy(
            sample_hbm.at[0, pl.ds(base, BPW)], hidx_all, sem_ih)
        ct = pltpu.async_copy(
            sample_hbm.at[1, pl.ds(base, BPW)], tidx_all, sem_it)
        ce = pltpu.async_copy(et_hbm.at[pl.ds(base, BPW)], eidx_all, sem_ir)
        ci.wait()
        ct.wait()
        ce.wait()

        bufs = (
            (head0, tail0, rel0, sem_h0, sem_t0, sem_r0),
            (head1, tail1, rel1, sem_h1, sem_t1, sem_r1),
        )

        def issue(c, b):
            head_v, tail_v, rel_v, sh, st, sr = bufs[b]
            co = c * CHUNK
            pltpu.async_copy(
                ent_hbm.at[hidx_all.at[pl.ds(co, CHUNK)]], head_v, sh)
            pltpu.async_copy(
                ent_hbm.at[tidx_all.at[pl.ds(co, CHUNK)]], tail_v, st)
            pltpu.async_copy(
                rel_hbm.at[eidx_all.at[pl.ds(co, CHUNK)]], rel_v, sr)

        def drain(b):
            head_v, tail_v, rel_v, sh, st, sr = bufs[b]
            pltpu.make_async_copy(
                ent_hbm.at[pl.ds(0, CHUNK)], head_v, sh).wait()
            pltpu.make_async_copy(
                ent_hbm.at[pl.ds(0, CHUNK)], tail_v, st).wait()
            pltpu.make_async_copy(
                rel_hbm.at[pl.ds(0, CHUNK)], rel_v, sr).wait()

        def compute(c, b):
            head_v, tail_v, rel_v, _, _, _ = bufs[b]
            cb = base + c * CHUNK

            def group_body(g, carry1):
                def elem_body(e, carry2):
                    i = g * 16 + e
                    acc = jnp.zeros((16,), jnp.float32)
                    for dc in range(NDC):
                        sl = pl.ds(dc * 16, 16)
                        sl_im = pl.ds(HIDDEN + dc * 16, 16)
                        cos_r = rel_v[i, sl]
                        sin_r = rel_v[i, sl_im]
                        re_h = head_v[i, sl]
                        im_h = head_v[i, sl_im]
                        re_s = re_h * cos_r - im_h * sin_r - tail_v[i, sl]
                        im_s = re_h * sin_r + im_h * cos_r - tail_v[i, sl_im]
                        acc = acc + _sqrt16(re_s * re_s + im_s * im_s)
                    plsc.store_scatter(red_v, [lane_iota + e * RED_STRIDE],
                                       acc)
                    return carry2

                lax.fori_loop(0, 16, elem_body, 0)
                tot = jnp.zeros((16,), jnp.float32)
                for cc in range(16):
                    tot = tot + plsc.load_gather(red_v, [col_base + cc])
                sc = GAMMA - tot
                u = jnp.exp(-jnp.abs(sc))
                t = u / (u + 2.0)
                t2 = t * t
                log1p = 2.0 * t * (1.0 + t2 * (1.0 / 3.0 + t2 * (
                    1.0 / 5.0 + t2 * (1.0 / 7.0))))
                out_v[pl.ds(g * 16, 16)] = jnp.minimum(sc, 0.0) - log1p
                return carry1

            lax.fori_loop(0, NGRP, group_body, 0)
            pltpu.sync_copy(out_v, out_hbm.at[pl.ds(cb, CHUNK)])

        issue(0, 0)

        def pair_body(k, carry):
            c0 = 2 * k
            issue(c0 + 1, 1)
            drain(0)
            compute(c0, 0)
            issue(c0 + 2, 0)
            drain(1)
            compute(c0 + 1, 1)
            return carry

        lax.fori_loop(0, NPAIR - 1, pair_body, 0)
        issue(NCHUNK - 1, 1)
        drain(0)
        compute(NCHUNK - 2, 0)
        drain(1)
        compute(NCHUNK - 1, 1)

    return rotate_score


_SC_KERNEL = _make_sc_kernel()


@jax.jit
def kernel(sample, et, entity_embedding, relation_embedding):
    sincos = _sincos_table(relation_embedding)
    return _SC_KERNEL(sample, et, entity_embedding, sincos)


# speedup vs baseline: 2.5830x; 1.0006x over previous
import functools

import jax
import jax.numpy as jnp
from jax import lax
from jax.experimental import pallas as pl
from jax.experimental.pallas import tpu as pltpu
from jax.experimental.pallas import tpu_sc as plsc

GAMMA = 12.0
HIDDEN = 128
ENT_DIM = 256
BATCH = 16384
EMB_RANGE = (12.0 + 2.0) / HIDDEN
PHASE_K = 3.141592653589793 / EMB_RANGE

NUM_WORKERS = 32
BPW = BATCH // NUM_WORKERS
CHUNK = 64
NCHUNK = BPW // CHUNK
NPAIR = NCHUNK // 2
NGRP = CHUNK // 16
NDC = HIDDEN // 16
RED_STRIDE = 17

NREL = 1000
RSQRT_MAGIC = 0x5F3759DF


def _sincos_tc_kernel(rel_ref, out_ref):
    ph = rel_ref[...] * PHASE_K
    out_ref[:, :HIDDEN] = jnp.cos(ph)
    out_ref[:, HIDDEN:] = jnp.sin(ph)


_sincos_table = pl.pallas_call(
    _sincos_tc_kernel,
    out_shape=jax.ShapeDtypeStruct((NREL, 2 * HIDDEN), jnp.float32),
)


def _sqrt16(v):
    bits = lax.bitcast_convert_type(v, jnp.int32)
    seed = RSQRT_MAGIC - lax.shift_right_arithmetic(bits, 1)
    y = lax.bitcast_convert_type(seed, jnp.float32)
    half_v = 0.5 * v
    for _ in range(2):
        y = y * (1.5 - half_v * y * y)
    return v * y


def _make_sc_kernel():
    mesh = plsc.VectorSubcoreMesh(core_axis_name="c", subcore_axis_name="s")

    @functools.partial(
        pl.kernel,
        mesh=mesh,
        compiler_params=pltpu.CompilerParams(needs_layout_passes=False),
        out_type=jax.ShapeDtypeStruct((BATCH,), jnp.float32),
        scratch_types=[
            pltpu.VMEM((BPW,), jnp.int32),
            pltpu.VMEM((BPW,), jnp.int32),
            pltpu.VMEM((BPW,), jnp.int32),
        ] + [
            pltpu.VMEM((CHUNK, ENT_DIM), jnp.float32)
            for _ in range(6)
        ] + [
            pltpu.VMEM((CHUNK,), jnp.float32),
            pltpu.VMEM((16 * RED_STRIDE,), jnp.float32),
            pltpu.SemaphoreType.DMA,
            pltpu.SemaphoreType.DMA,
            pltpu.SemaphoreType.DMA,
        ] + [
            pltpu.SemaphoreType.DMA
            for _ in range(6)
        ],
    )
    def rotate_score(sample_hbm, et_hbm, ent_hbm, rel_hbm, out_hbm,
                     hidx_all, tidx_all, eidx_all,
                     head0, tail0, rel0, head1, tail1, rel1,
                     out_v, red_v, sem_ih, sem_it, sem_ir,
                     sem_h0, sem_t0, sem_r0, sem_h1, sem_t1, sem_r1):
        wid = lax.axis_index("s") * 2 + lax.axis_index("c")
        base = wid * BPW
        lane_iota = lax.broadcasted_iota(jnp.int32, (16,), 0)
        col_base = lane_iota * RED_STRIDE

        ci = pltpu.async_copy(
            sample_hbm.at[0, pl.ds(base, BPW)], hidx_all, sem_ih)
        ct = pltpu.async_copy(
            sample_hbm.at[1, pl.ds(base, BPW)], tidx_all, sem_it)
        ce = pltpu.async_copy(et_hbm.at[pl.ds(base, BPW)], eidx_all, sem_ir)
        ci.wait()
        ct.wait()
        ce.wait()

        bufs = (
            (head0, tail0, rel0, sem_h0, sem_t0, sem_r0),
            (head1, tail1, rel1, sem_h1, sem_t1, sem_r1),
        )

        def issue(c, b):
            head_v, tail_v, rel_v, sh, st, sr = bufs[b]
            co = c * CHUNK
            pltpu.async_copy(
                ent_hbm.at[hidx_all.at[pl.ds(co, CHUNK)]], head_v, sh)
            pltpu.async_copy(
                ent_hbm.at[tidx_all.at[pl.ds(co, CHUNK)]], tail_v, st)
            pltpu.async_copy(
                rel_hbm.at[eidx_all.at[pl.ds(co, CHUNK)]], rel_v, sr)

        def drain(b):
            head_v, tail_v, rel_v, sh, st, sr = bufs[b]
            pltpu.make_async_copy(
                ent_hbm.at[pl.ds(0, CHUNK)], head_v, sh).wait()
            pltpu.make_async_copy(
                ent_hbm.at[pl.ds(0, CHUNK)], tail_v, st).wait()
            pltpu.make_async_copy(
                rel_hbm.at[pl.ds(0, CHUNK)], rel_v, sr).wait()

        def compute(c, b):
            head_v, tail_v, rel_v, _, _, _ = bufs[b]
            cb = base + c * CHUNK

            def group_body(g, carry1):
                def elem_body(e, carry2):
                    i = g * 16 + e
                    acc = jnp.zeros((16,), jnp.float32)
                    for dc in range(NDC):
                        sl = pl.ds(dc * 16, 16)
                        sl_im = pl.ds(HIDDEN + dc * 16, 16)
                        cos_r = rel_v[i, sl]
                        sin_r = rel_v[i, sl_im]
                        re_h = head_v[i, sl]
                        im_h = head_v[i, sl_im]
                        re_s = re_h * cos_r - im_h * sin_r - tail_v[i, sl]
                        im_s = re_h * sin_r + im_h * cos_r - tail_v[i, sl_im]
                        acc = acc + _sqrt16(re_s * re_s + im_s * im_s)
                    plsc.store_scatter(red_v, [lane_iota + e * RED_STRIDE],
                                       acc)
                    return carry2

                lax.fori_loop(0, 16, elem_body, 0)
                tot = jnp.zeros((16,), jnp.float32)
                for cc in range(16):
                    tot = tot + plsc.load_gather(red_v, [col_base + cc])
                sc = GAMMA - tot
                u = jnp.exp(-jnp.abs(sc))
                t = u / (u + 2.0)
                t2 = t * t
                log1p = 2.0 * t * (1.0 + t2 * (1.0 / 3.0 + t2 * (
                    1.0 / 5.0 + t2 * (1.0 / 7.0))))
                out_v[pl.ds(g * 16, 16)] = jnp.minimum(sc, 0.0) - log1p
                return carry1

            lax.fori_loop(0, NGRP, group_body, 0)
            pltpu.sync_copy(out_v, out_hbm.at[pl.ds(cb, CHUNK)])

        issue(0, 0)

        def pair_body(k, carry):
            c0 = 2 * k
            issue(c0 + 1, 1)
            drain(0)
            compute(c0, 0)
            issue(c0 + 2, 0)
            drain(1)
            compute(c0 + 1, 1)
            return carry

        lax.fori_loop(0, NPAIR - 1, pair_body, 0)
        issue(NCHUNK - 1, 1)
        drain(0)
        compute(NCHUNK - 2, 0)
        drain(1)
        compute(NCHUNK - 1, 1)

    return rotate_score


_SC_KERNEL = _make_sc_kernel()


@jax.jit
def kernel(sample, et, entity_embedding, relation_embedding):
    sincos = _sincos_table(relation_embedding)
    return _SC_KERNEL(sample, et, entity_embedding, sincos)
